# Initial kernel scaffold; baseline (speedup 1.0000x reference)
import jax
import jax.numpy as jnp
from jax.experimental import pallas as pl


def kernel(x, prev_input, prev_output, weight, bias):
    raise NotImplementedError("write your pallas kernel here")


# speedup vs baseline: 66.2989x; 66.2989x over previous
import jax
import jax.numpy as jnp
from jax.experimental import pallas as pl
from jax.experimental.pallas import tpu as pltpu
from functools import partial

_THRESHOLD = 5.0
_KH, _KW = 3, 3


def _cbconv_body(xf_ref, pif_ref, pof_ref, wm_ref, b2_ref, out_ref,
                 xpad_ref, cpad_ref, patch_ref,
                 *, C, Cout, H, W, Rb, PAD):
    HW = H * W
    Nb = Rb * W
    rb = pl.program_id(1)

    @pl.when(rb == 0)
    def _init():
        zc = jnp.zeros((C, PAD), jnp.float32)
        xpad_ref[:, 0:PAD] = zc
        xpad_ref[:, PAD + HW:] = zc
        xpad_ref[:, PAD:PAD + HW] = xf_ref[0]
        z1 = jnp.zeros((1, PAD), jnp.float32)
        cpad_ref[:, 0:PAD] = z1
        cpad_ref[:, PAD + HW:] = z1
        ch = jnp.any(jnp.abs(xf_ref[0] - pif_ref[0]) > _THRESHOLD,
                     axis=0, keepdims=True)
        cpad_ref[:, PAD:PAD + HW] = ch.astype(jnp.float32)

    base = PAD + rb * Nb

    lane = jax.lax.broadcasted_iota(jnp.int32, (1, Nb), 1) % W
    mleft = (lane != 0).astype(jnp.float32)
    mright = (lane != (W - 1)).astype(jnp.float32)

    for dh in range(3):
        xw = xpad_ref[:, pl.ds(base + (dh - 1) * W - 128, Nb + 256)]
        for dw in range(3):
            k = dh * 3 + dw
            sl = jax.lax.slice(xw, (0, 128 + dw - 1), (C, 128 + dw - 1 + Nb))
            if dw == 0:
                sl = sl * mleft
            elif dw == 2:
                sl = sl * mright
            patch_ref[k * C:(k + 1) * C, :] = sl

    y = jnp.dot(wm_ref[:, :], patch_ref[:, :],
                preferred_element_type=jnp.float32) + b2_ref[:, :]

    dil = None
    for dr in range(3):
        cw = cpad_ref[:, pl.ds(base + (dr - 1) * W - 128, Nb + 256)]
        for dc in range(3):
            s = jax.lax.slice(cw, (0, 128 + dc - 1), (1, 128 + dc - 1 + Nb))
            if dc == 0:
                s = s * mleft
            elif dc == 2:
                s = s * mright
            dil = s if dil is None else jnp.maximum(dil, s)

    out_ref[0] = jnp.where(dil > 0.0, y, pof_ref[0])


def kernel(x, prev_input, prev_output, weight, bias):
    B, C, H, W = x.shape
    Cout = weight.shape[0]
    HW = H * W
    Rb = 16
    NB = H // Rb
    Nb = Rb * W
    PAD = 256

    xf = x.reshape(B, C, HW)
    pif = prev_input.reshape(B, C, HW)
    pof = prev_output.reshape(B, Cout, HW)
    wm = jnp.transpose(weight, (0, 2, 3, 1)).reshape(Cout, _KH * _KW * C)
    b2 = bias.reshape(Cout, 1)

    body = partial(_cbconv_body, C=C, Cout=Cout, H=H, W=W, Rb=Rb, PAD=PAD)
    out = pl.pallas_call(
        body,
        grid=(B, NB),
        in_specs=[
            pl.BlockSpec((1, C, HW), lambda b, rb: (b, 0, 0)),
            pl.BlockSpec((1, C, HW), lambda b, rb: (b, 0, 0)),
            pl.BlockSpec((1, Cout, Nb), lambda b, rb: (b, 0, rb)),
            pl.BlockSpec((Cout, _KH * _KW * C), lambda b, rb: (0, 0)),
            pl.BlockSpec((Cout, 1), lambda b, rb: (0, 0)),
        ],
        out_specs=pl.BlockSpec((1, Cout, Nb), lambda b, rb: (b, 0, rb)),
        out_shape=jax.ShapeDtypeStruct((B, Cout, HW), jnp.float32),
        scratch_shapes=[
            pltpu.VMEM((C, PAD + HW + PAD), jnp.float32),
            pltpu.VMEM((1, PAD + HW + PAD), jnp.float32),
            pltpu.VMEM((_KH * _KW * C, Nb), jnp.float32),
        ],
    )(xf, pif, pof, wm, b2)
    return out.reshape(B, Cout, H, W)


# speedup vs baseline: 74.5863x; 1.1250x over previous
import jax
import jax.numpy as jnp
from jax.experimental import pallas as pl
from jax.experimental.pallas import tpu as pltpu
from functools import partial

_THRESHOLD = 5.0
_KH, _KW = 3, 3


def _cbconv_body(xf_ref, pif_ref, pof_ref, wf_ref, br_ref, out_ref,
                 xpad_ref, cpad_ref, rhs_ref, l_ref, bcol_ref,
                 *, C, Cout, H, W, Rb, PAD):
    HW = H * W
    Nb = Rb * W
    NW = Nb + 256
    b = pl.program_id(0)
    rb = pl.program_id(1)

    @pl.when(jnp.logical_and(b == 0, rb == 0))
    def _init_call():
        f_ids = jax.lax.broadcasted_iota(jnp.int32, (9 * C, 3 * C), 0)
        s_ids = jax.lax.broadcasted_iota(jnp.int32, (9 * C, 3 * C), 1)
        f_target = (s_ids % C) * 9 + (s_ids // C) * 3
        for dw in range(3):
            q = (f_ids == f_target + dw).astype(jnp.float32)
            l_ref[dw * Cout:(dw + 1) * Cout, :] = jnp.dot(
                wf_ref[:, :], q, preferred_element_type=jnp.float32)
        bcol_ref[:, :] = jnp.transpose(br_ref[:, :], (1, 0))

    @pl.when(rb == 0)
    def _init_batch():
        zc = jnp.zeros((C, PAD), jnp.float32)
        xpad_ref[:, 0:PAD] = zc
        xpad_ref[:, PAD + HW:] = zc
        xpad_ref[:, PAD:PAD + HW] = xf_ref[0]
        z1 = jnp.zeros((1, PAD), jnp.float32)
        cpad_ref[:, 0:PAD] = z1
        cpad_ref[:, PAD + HW:] = z1
        m = jnp.max(jnp.abs(xf_ref[0] - pif_ref[0]), axis=0, keepdims=True)
        cpad_ref[:, PAD:PAD + HW] = (m > _THRESHOLD).astype(jnp.float32)

    lane = jax.lax.broadcasted_iota(jnp.int32, (1, Nb), 1) % W
    mleft = (lane != 0).astype(jnp.float32)
    mright = (lane != (W - 1)).astype(jnp.float32)

    base = PAD + rb * Nb
    for dh in range(3):
        rhs_ref[dh * C:(dh + 1) * C, :] = \
            xpad_ref[:, pl.ds(base + (dh - 1) * W - 128, NW)]

    z = jnp.dot(l_ref[:, :], rhs_ref[:, :],
                preferred_element_type=jnp.float32)
    y = (jax.lax.slice(z, (0, 127), (Cout, 127 + Nb)) * mleft
         + jax.lax.slice(z, (Cout, 128), (2 * Cout, 128 + Nb))
         + jax.lax.slice(z, (2 * Cout, 129), (3 * Cout, 129 + Nb)) * mright
         + bcol_ref[:, :])

    dil = None
    for dr in range(3):
        cw = cpad_ref[:, pl.ds(base + (dr - 1) * W - 128, NW)]
        for dc in range(3):
            s = jax.lax.slice(cw, (0, 128 + dc - 1),
                              (1, 128 + dc - 1 + Nb))
            if dc == 0:
                s = s * mleft
            elif dc == 2:
                s = s * mright
            dil = s if dil is None else jnp.maximum(dil, s)

    out_ref[0] = jnp.where(dil > 0.0, y, pof_ref[0])


def kernel(x, prev_input, prev_output, weight, bias):
    B, C, H, W = x.shape
    Cout = weight.shape[0]
    HW = H * W
    Rb = 16
    NB = H // Rb
    Nb = Rb * W
    PAD = 256

    xf = x.reshape(B, C, HW)
    pif = prev_input.reshape(B, C, HW)
    pof = prev_output.reshape(B, Cout, HW)
    wf = weight.reshape(Cout, C * _KH * _KW)
    br = bias.reshape(1, Cout)

    body = partial(_cbconv_body, C=C, Cout=Cout, H=H, W=W, Rb=Rb, PAD=PAD)
    out = pl.pallas_call(
        body,
        grid=(B, NB),
        in_specs=[
            pl.BlockSpec((1, C, HW), lambda b, rb: (b, 0, 0)),
            pl.BlockSpec((1, C, HW), lambda b, rb: (b, 0, 0)),
            pl.BlockSpec((1, Cout, Nb), lambda b, rb: (b, 0, rb)),
            pl.BlockSpec((Cout, _KH * _KW * C), lambda b, rb: (0, 0)),
            pl.BlockSpec((1, Cout), lambda b, rb: (0, 0)),
        ],
        out_specs=pl.BlockSpec((1, Cout, Nb), lambda b, rb: (b, 0, rb)),
        out_shape=jax.ShapeDtypeStruct((B, Cout, HW), jnp.float32),
        scratch_shapes=[
            pltpu.VMEM((C, PAD + HW + PAD), jnp.float32),
            pltpu.VMEM((1, PAD + HW + PAD), jnp.float32),
            pltpu.VMEM((3 * C, Nb + PAD), jnp.float32),
            pltpu.VMEM((3 * Cout, 3 * C), jnp.float32),
            pltpu.VMEM((Cout, 1), jnp.float32),
        ],
    )(xf, pif, pof, wf, br)
    return out.reshape(B, Cout, H, W)


# speedup vs baseline: 114.9845x; 1.5416x over previous
import jax
import jax.numpy as jnp
from jax.experimental import pallas as pl
from jax.experimental.pallas import tpu as pltpu
from functools import partial

_THRESHOLD = 5.0
_KH, _KW = 3, 3


def _cbconv_body(x_ref, pi_ref, po_ref, wf_ref, br_ref, out_ref,
                 xpad_ref, cpad_ref, rhs_ref, l_ref, bcol_ref,
                 *, C, Cout, H, W, Rb, PAD):
    HW = H * W
    Nb = Rb * W
    NW = Nb + 256
    b = pl.program_id(0)
    rb = pl.program_id(1)

    @pl.when(jnp.logical_and(b == 0, rb == 0))
    def _init_call():
        f_ids = jax.lax.broadcasted_iota(jnp.int32, (9 * C, 3 * C), 0)
        s_ids = jax.lax.broadcasted_iota(jnp.int32, (9 * C, 3 * C), 1)
        f_target = (s_ids % C) * 9 + (s_ids // C) * 3
        for dw in range(3):
            q = (f_ids == f_target + dw).astype(jnp.float32)
            l_ref[dw * Cout:(dw + 1) * Cout, :] = jnp.dot(
                wf_ref[:, :], q, preferred_element_type=jnp.float32)
        bcol_ref[:, :] = jnp.transpose(br_ref[:, :], (1, 0))

    @pl.when(rb == 0)
    def _init_batch():
        zc = jnp.zeros((C, PAD), jnp.float32)
        xpad_ref[:, 0:PAD] = zc
        xpad_ref[:, PAD + HW:] = zc
        xpad_ref[:, PAD:PAD + HW] = x_ref[0].reshape(C, HW)
        z1 = jnp.zeros((1, PAD), jnp.float32)
        cpad_ref[:, 0:PAD] = z1
        cpad_ref[:, PAD + HW:] = z1
        m = jnp.max(jnp.abs(x_ref[0] - pi_ref[0]), axis=0)
        cpad_ref[:, PAD:PAD + HW] = \
            (m > _THRESHOLD).astype(jnp.float32).reshape(1, HW)

    lane = jax.lax.broadcasted_iota(jnp.int32, (1, Nb), 1) % W
    mleft = (lane != 0).astype(jnp.float32)
    mright = (lane != (W - 1)).astype(jnp.float32)

    base = PAD + rb * Nb
    for dh in range(3):
        rhs_ref[dh * C:(dh + 1) * C, :] = \
            xpad_ref[:, pl.ds(base + (dh - 1) * W - 128, NW)]

    z = jnp.dot(l_ref[:, :], rhs_ref[:, :],
                preferred_element_type=jnp.float32)
    y = (jax.lax.slice(z, (0, 127), (Cout, 127 + Nb)) * mleft
         + jax.lax.slice(z, (Cout, 128), (2 * Cout, 128 + Nb))
         + jax.lax.slice(z, (2 * Cout, 129), (3 * Cout, 129 + Nb)) * mright
         + bcol_ref[:, :])

    dil = None
    for dr in range(3):
        cw = cpad_ref[:, pl.ds(base + (dr - 1) * W - 128, NW)]
        for dc in range(3):
            s = jax.lax.slice(cw, (0, 128 + dc - 1),
                              (1, 128 + dc - 1 + Nb))
            if dc == 0:
                s = s * mleft
            elif dc == 2:
                s = s * mright
            dil = s if dil is None else jnp.maximum(dil, s)

    out_ref[0] = jnp.where(dil > 0.0, y, po_ref[0].reshape(Cout, Nb))


def kernel(x, prev_input, prev_output, weight, bias):
    B, C, H, W = x.shape
    Cout = weight.shape[0]
    HW = H * W
    Rb = 16
    NB = H // Rb
    Nb = Rb * W
    PAD = 256

    wf = weight.reshape(Cout, C * _KH * _KW)
    br = bias.reshape(1, Cout)

    body = partial(_cbconv_body, C=C, Cout=Cout, H=H, W=W, Rb=Rb, PAD=PAD)
    out = pl.pallas_call(
        body,
        grid=(B, NB),
        in_specs=[
            pl.BlockSpec((1, C, H, W), lambda b, rb: (b, 0, 0, 0)),
            pl.BlockSpec((1, C, H, W), lambda b, rb: (b, 0, 0, 0)),
            pl.BlockSpec((1, Cout, Rb, W), lambda b, rb: (b, 0, rb, 0)),
            pl.BlockSpec((Cout, _KH * _KW * C), lambda b, rb: (0, 0)),
            pl.BlockSpec((1, Cout), lambda b, rb: (0, 0)),
        ],
        out_specs=pl.BlockSpec((1, Cout, Nb), lambda b, rb: (b, 0, rb)),
        out_shape=jax.ShapeDtypeStruct((B, Cout, HW), jnp.float32),
        scratch_shapes=[
            pltpu.VMEM((C, PAD + HW + PAD), jnp.float32),
            pltpu.VMEM((1, PAD + HW + PAD), jnp.float32),
            pltpu.VMEM((3 * C, Nb + 256), jnp.float32),
            pltpu.VMEM((3 * Cout, 3 * C), jnp.float32),
            pltpu.VMEM((Cout, 1), jnp.float32),
        ],
    )(x, prev_input, prev_output, wf, br)
    return out.reshape(B, Cout, H, W)


# speedup vs baseline: 193.7150x; 1.6847x over previous
import jax
import jax.numpy as jnp
from jax.experimental import pallas as pl
from jax.experimental.pallas import tpu as pltpu
from functools import partial

_THRESHOLD = 5.0
_KH, _KW = 3, 3


def _cbconv_body(x_ref, pi_ref, po_ref, wf_ref, br_ref, out_ref,
                 xpad_ref, cpad_ref, rhs_ref, l_ref, bcol_ref,
                 *, C, Cout, H, W, Rb, PAD):
    HW = H * W
    Nb = Rb * W
    NW = Nb + 256
    b = pl.program_id(0)
    rb = pl.program_id(1)

    @pl.when(jnp.logical_and(b == 0, rb == 0))
    def _init_call():
        f_ids = jax.lax.broadcasted_iota(jnp.int32, (9 * C, 3 * C), 0)
        s_ids = jax.lax.broadcasted_iota(jnp.int32, (9 * C, 3 * C), 1)
        f_target = (s_ids % C) * 9 + (s_ids // C) * 3
        for dw in range(3):
            q = (f_ids == f_target + dw).astype(jnp.float32)
            l_ref[dw * Cout:(dw + 1) * Cout, :] = jnp.dot(
                wf_ref[:, :], q, preferred_element_type=jnp.float32)
        bcol_ref[:, :] = jnp.transpose(br_ref[:, :], (1, 0))

    @pl.when(rb == 0)
    def _init_batch():
        zc = jnp.zeros((C, PAD), jnp.float32)
        xpad_ref[:, 0:PAD] = zc
        xpad_ref[:, PAD + HW:] = zc
        xpad_ref[:, PAD:PAD + HW] = x_ref[0].reshape(C, HW)
        z1 = jnp.zeros((1, PAD), jnp.float32)
        cpad_ref[:, 0:PAD] = z1
        cpad_ref[:, PAD + HW:] = z1
        m = jnp.max(jnp.abs(x_ref[0] - pi_ref[0]), axis=0)
        cpad_ref[:, PAD:PAD + HW] = \
            (m > _THRESHOLD).astype(jnp.float32).reshape(1, HW)

    lane = jax.lax.broadcasted_iota(jnp.int32, (1, Nb), 1) % W
    mleft = (lane != 0).astype(jnp.float32)
    mright = (lane != (W - 1)).astype(jnp.float32)

    base = PAD + rb * Nb
    for dh in range(3):
        rhs_ref[dh * C:(dh + 1) * C, :] = \
            xpad_ref[:, pl.ds(base + (dh - 1) * W - 128, NW)]

    z = jnp.dot(l_ref[:, :], rhs_ref[:, :],
                preferred_element_type=jnp.float32)
    y = (jax.lax.slice(z, (0, 127), (Cout, 127 + Nb)) * mleft
         + jax.lax.slice(z, (Cout, 128), (2 * Cout, 128 + Nb))
         + jax.lax.slice(z, (2 * Cout, 129), (3 * Cout, 129 + Nb)) * mright
         + bcol_ref[:, :])

    dil = None
    for dr in range(3):
        cw = cpad_ref[:, pl.ds(base + (dr - 1) * W - 128, NW)]
        for dc in range(3):
            s = jax.lax.slice(cw, (0, 128 + dc - 1),
                              (1, 128 + dc - 1 + Nb))
            if dc == 0:
                s = s * mleft
            elif dc == 2:
                s = s * mright
            dil = s if dil is None else jnp.maximum(dil, s)

    sel = jnp.where(dil > 0.0, y, po_ref[0].reshape(Cout, Nb))
    for t in range(Rb // 8):
        out_ref[0, :, t * 8:(t + 1) * 8, :] = jax.lax.slice(
            sel, (0, t * 8 * W), (Cout, (t + 1) * 8 * W)).reshape(Cout, 8, W)


def kernel(x, prev_input, prev_output, weight, bias):
    B, C, H, W = x.shape
    Cout = weight.shape[0]
    HW = H * W
    Rb = 16
    NB = H // Rb
    Nb = Rb * W
    PAD = 256

    wf = weight.reshape(Cout, C * _KH * _KW)
    br = bias.reshape(1, Cout)

    body = partial(_cbconv_body, C=C, Cout=Cout, H=H, W=W, Rb=Rb, PAD=PAD)
    out = pl.pallas_call(
        body,
        grid=(B, NB),
        in_specs=[
            pl.BlockSpec((1, C, H, W), lambda b, rb: (b, 0, 0, 0)),
            pl.BlockSpec((1, C, H, W), lambda b, rb: (b, 0, 0, 0)),
            pl.BlockSpec((1, Cout, Rb, W), lambda b, rb: (b, 0, rb, 0)),
            pl.BlockSpec((Cout, _KH * _KW * C), lambda b, rb: (0, 0)),
            pl.BlockSpec((1, Cout), lambda b, rb: (0, 0)),
        ],
        out_specs=pl.BlockSpec((1, Cout, Rb, W), lambda b, rb: (b, 0, rb, 0)),
        out_shape=jax.ShapeDtypeStruct((B, Cout, H, W), jnp.float32),
        scratch_shapes=[
            pltpu.VMEM((C, PAD + HW + PAD), jnp.float32),
            pltpu.VMEM((1, PAD + HW + PAD), jnp.float32),
            pltpu.VMEM((3 * C, Nb + 256), jnp.float32),
            pltpu.VMEM((3 * Cout, 3 * C), jnp.float32),
            pltpu.VMEM((Cout, 1), jnp.float32),
        ],
    )(x, prev_input, prev_output, wf, br)
    return out


# speedup vs baseline: 212.0435x; 1.0946x over previous
import jax
import jax.numpy as jnp
from jax.experimental import pallas as pl
from jax.experimental.pallas import tpu as pltpu
from functools import partial

_THRESHOLD = 5.0
_KH, _KW = 3, 3


def _cbconv_body(x_ref, pi_ref, po_ref, wf_ref, br_ref, out_ref,
                 xpad_ref, cpad_ref, rhs_ref, l_ref, bcol_ref,
                 *, C, Cout, H, W, Rb, PAD, NB):
    HW = H * W
    Nb = Rb * W
    NW = Nb + 256
    b = pl.program_id(0)
    j = pl.program_id(1)

    @pl.when(jnp.logical_and(b == 0, j == 0))
    def _init_call():
        f_ids = jax.lax.broadcasted_iota(jnp.int32, (9 * C, 3 * C), 0)
        s_ids = jax.lax.broadcasted_iota(jnp.int32, (9 * C, 3 * C), 1)
        f_target = (s_ids % C) * 9 + (s_ids // C) * 3
        for dw in range(3):
            q = (f_ids == f_target + dw).astype(jnp.float32)
            l_ref[dw * Cout:(dw + 1) * Cout, :] = jnp.dot(
                wf_ref[:, :], q, preferred_element_type=jnp.float32)
        bcol_ref[:, :] = jnp.transpose(br_ref[:, :], (1, 0))
        xpad_ref[:, 0:PAD] = jnp.zeros((C, PAD), jnp.float32)
        xpad_ref[:, PAD + HW:] = jnp.zeros((C, PAD), jnp.float32)
        z1 = jnp.zeros((1, PAD), jnp.float32)
        cpad_ref[:, 0:PAD] = z1
        cpad_ref[:, PAD + HW:] = z1

    @pl.when(j < NB)
    def _stage_chunk():
        xc = x_ref[0]
        xpad_ref[:, pl.ds(PAD + j * Nb, Nb)] = xc.reshape(C, Nb)
        m = jnp.max(jnp.abs(xc - pi_ref[0]), axis=0)
        cpad_ref[:, pl.ds(PAD + j * Nb, Nb)] = \
            (m > _THRESHOLD).astype(jnp.float32).reshape(1, Nb)

    @pl.when(j > 0)
    def _compute_block():
        rb = j - 1
        lane = jax.lax.broadcasted_iota(jnp.int32, (1, Nb), 1) % W
        mleft = (lane != 0).astype(jnp.float32)
        mright = (lane != (W - 1)).astype(jnp.float32)

        base = PAD + rb * Nb
        for dh in range(3):
            rhs_ref[dh * C:(dh + 1) * C, :] = \
                xpad_ref[:, pl.ds(base + (dh - 1) * W - 128, NW)]

        z = jnp.dot(l_ref[:, :], rhs_ref[:, :],
                    preferred_element_type=jnp.float32)
        y = (jax.lax.slice(z, (0, 127), (Cout, 127 + Nb)) * mleft
             + jax.lax.slice(z, (Cout, 128), (2 * Cout, 128 + Nb))
             + jax.lax.slice(z, (2 * Cout, 129), (3 * Cout, 129 + Nb))
             * mright
             + bcol_ref[:, :])

        dil = None
        for dr in range(3):
            cw = cpad_ref[:, pl.ds(base + (dr - 1) * W - 128, NW)]
            for dc in range(3):
                s = jax.lax.slice(cw, (0, 128 + dc - 1),
                                  (1, 128 + dc - 1 + Nb))
                if dc == 0:
                    s = s * mleft
                elif dc == 2:
                    s = s * mright
                dil = s if dil is None else jnp.maximum(dil, s)

        sel = jnp.where(dil > 0.0, y, po_ref[0].reshape(Cout, Nb))
        for t in range(Rb // 8):
            out_ref[0, :, t * 8:(t + 1) * 8, :] = jax.lax.slice(
                sel, (0, t * 8 * W), (Cout, (t + 1) * 8 * W)
            ).reshape(Cout, 8, W)


def kernel(x, prev_input, prev_output, weight, bias):
    B, C, H, W = x.shape
    Cout = weight.shape[0]
    HW = H * W
    Rb = 16
    NB = H // Rb
    Nb = Rb * W
    PAD = 256

    wf = weight.reshape(Cout, C * _KH * _KW)
    br = bias.reshape(1, Cout)

    body = partial(_cbconv_body, C=C, Cout=Cout, H=H, W=W, Rb=Rb, PAD=PAD,
                   NB=NB)
    last = NB - 1
    out = pl.pallas_call(
        body,
        grid=(B, NB + 1),
        in_specs=[
            pl.BlockSpec((1, C, Rb, W),
                         lambda b, j: (b, 0, jnp.minimum(j, last), 0)),
            pl.BlockSpec((1, C, Rb, W),
                         lambda b, j: (b, 0, jnp.minimum(j, last), 0)),
            pl.BlockSpec((1, Cout, Rb, W),
                         lambda b, j: (b, 0, jnp.maximum(j - 1, 0), 0)),
            pl.BlockSpec((Cout, _KH * _KW * C), lambda b, j: (0, 0)),
            pl.BlockSpec((1, Cout), lambda b, j: (0, 0)),
        ],
        out_specs=pl.BlockSpec((1, Cout, Rb, W),
                               lambda b, j: (b, 0, jnp.maximum(j - 1, 0), 0)),
        out_shape=jax.ShapeDtypeStruct((B, Cout, H, W), jnp.float32),
        scratch_shapes=[
            pltpu.VMEM((C, PAD + HW + PAD), jnp.float32),
            pltpu.VMEM((1, PAD + HW + PAD), jnp.float32),
            pltpu.VMEM((3 * C, Nb + 256), jnp.float32),
            pltpu.VMEM((3 * Cout, 3 * C), jnp.float32),
            pltpu.VMEM((Cout, 1), jnp.float32),
        ],
    )(x, prev_input, prev_output, wf, br)
    return out


# speedup vs baseline: 225.8644x; 1.0652x over previous
import jax
import jax.numpy as jnp
from jax.experimental import pallas as pl
from jax.experimental.pallas import tpu as pltpu
from functools import partial

_THRESHOLD = 5.0
_KH, _KW = 3, 3


def _cbconv_body(x_ref, pi_ref, po_ref, wf_ref, br_ref, out_ref,
                 xpad_ref, cpad_ref, rhs_ref, l_ref, bcol_ref,
                 *, C, Cout, H, W, Rb, PAD, NB):
    HW = H * W
    Nb = Rb * W
    NW = Nb + 256
    b = pl.program_id(0)
    j = pl.program_id(1)

    @pl.when(jnp.logical_and(b == 0, j == 0))
    def _init_call():
        f_ids = jax.lax.broadcasted_iota(jnp.int32, (9 * C, 3 * C), 0)
        s_ids = jax.lax.broadcasted_iota(jnp.int32, (9 * C, 3 * C), 1)
        f_target = (s_ids % C) * 9 + (s_ids // C) * 3
        for dw in range(3):
            q = (f_ids == f_target + dw).astype(jnp.float32)
            l_ref[dw * Cout:(dw + 1) * Cout, :] = jnp.dot(
                wf_ref[:, :], q, preferred_element_type=jnp.float32)
        bcol_ref[:, :] = jnp.transpose(br_ref[:, :], (1, 0))
        xpad_ref[:, 0:PAD] = jnp.zeros((C, PAD), jnp.float32)
        xpad_ref[:, PAD + HW:] = jnp.zeros((C, PAD), jnp.float32)
        z1 = jnp.zeros((1, PAD), jnp.float32)
        cpad_ref[:, 0:PAD] = z1
        cpad_ref[:, PAD + HW:] = z1

    @pl.when(j < NB)
    def _stage_chunk():
        xc = x_ref[0]
        xpad_ref[:, pl.ds(PAD + j * Nb, Nb)] = xc.reshape(C, Nb)
        m = jnp.max(jnp.abs(xc - pi_ref[0]), axis=0)
        cpad_ref[:, pl.ds(PAD + j * Nb, Nb)] = \
            (m > _THRESHOLD).astype(jnp.float32).reshape(1, Nb)

    @pl.when(j > 0)
    def _compute_block():
        rb = j - 1
        lane = jax.lax.broadcasted_iota(jnp.int32, (1, Nb), 1) % W
        mleft = (lane != 0).astype(jnp.float32)
        mright = (lane != (W - 1)).astype(jnp.float32)

        base = PAD + rb * Nb
        for dh in range(3):
            rhs_ref[dh * C:(dh + 1) * C, :] = \
                xpad_ref[:, pl.ds(base + (dh - 1) * W - 128, NW)]

        z = jnp.dot(l_ref[:, :], rhs_ref[:, :],
                    preferred_element_type=jnp.float32)
        y = (jax.lax.slice(z, (0, 127), (Cout, 127 + Nb)) * mleft
             + jax.lax.slice(z, (Cout, 128), (2 * Cout, 128 + Nb))
             + jax.lax.slice(z, (2 * Cout, 129), (3 * Cout, 129 + Nb))
             * mright
             + bcol_ref[:, :])

        dil = None
        for dr in range(3):
            cw = cpad_ref[:, pl.ds(base + (dr - 1) * W - 128, NW)]
            for dc in range(3):
                s = jax.lax.slice(cw, (0, 128 + dc - 1),
                                  (1, 128 + dc - 1 + Nb))
                if dc == 0:
                    s = s * mleft
                elif dc == 2:
                    s = s * mright
                dil = s if dil is None else jnp.maximum(dil, s)

        sel = jnp.where(dil > 0.0, y, po_ref[0].reshape(Cout, Nb))
        for t in range(Rb // 8):
            out_ref[0, :, t * 8:(t + 1) * 8, :] = jax.lax.slice(
                sel, (0, t * 8 * W), (Cout, (t + 1) * 8 * W)
            ).reshape(Cout, 8, W)


def kernel(x, prev_input, prev_output, weight, bias):
    B, C, H, W = x.shape
    Cout = weight.shape[0]
    HW = H * W
    Rb = 32
    NB = H // Rb
    Nb = Rb * W
    PAD = 256

    wf = weight.reshape(Cout, C * _KH * _KW)
    br = bias.reshape(1, Cout)

    body = partial(_cbconv_body, C=C, Cout=Cout, H=H, W=W, Rb=Rb, PAD=PAD,
                   NB=NB)
    last = NB - 1
    out = pl.pallas_call(
        body,
        grid=(B, NB + 1),
        in_specs=[
            pl.BlockSpec((1, C, Rb, W),
                         lambda b, j: (b, 0, jnp.minimum(j, last), 0)),
            pl.BlockSpec((1, C, Rb, W),
                         lambda b, j: (b, 0, jnp.minimum(j, last), 0)),
            pl.BlockSpec((1, Cout, Rb, W),
                         lambda b, j: (b, 0, jnp.maximum(j - 1, 0), 0)),
            pl.BlockSpec((Cout, _KH * _KW * C), lambda b, j: (0, 0)),
            pl.BlockSpec((1, Cout), lambda b, j: (0, 0)),
        ],
        out_specs=pl.BlockSpec((1, Cout, Rb, W),
                               lambda b, j: (b, 0, jnp.maximum(j - 1, 0), 0)),
        out_shape=jax.ShapeDtypeStruct((B, Cout, H, W), jnp.float32),
        scratch_shapes=[
            pltpu.VMEM((C, PAD + HW + PAD), jnp.float32),
            pltpu.VMEM((1, PAD + HW + PAD), jnp.float32),
            pltpu.VMEM((3 * C, Nb + 256), jnp.float32),
            pltpu.VMEM((3 * Cout, 3 * C), jnp.float32),
            pltpu.VMEM((Cout, 1), jnp.float32),
        ],
    )(x, prev_input, prev_output, wf, br)
    return out


# speedup vs baseline: 229.0614x; 1.0142x over previous
import jax
import jax.numpy as jnp
from jax.experimental import pallas as pl
from jax.experimental.pallas import tpu as pltpu
from functools import partial

_THRESHOLD = 5.0
_KH, _KW = 3, 3


def _cbconv_body(x_ref, pi_ref, po_ref, wf_ref, br_ref, out_ref,
                 xpad_ref, cpad_ref, rhs_ref, l_ref, bcol_ref,
                 *, C, Cout, H, W, Rb, PAD, NB):
    HW = H * W
    Nb = Rb * W
    NW = Nb + 256
    b = pl.program_id(0)
    j = pl.program_id(1)

    @pl.when(jnp.logical_and(b == 0, j == 0))
    def _init_call():
        f_ids = jax.lax.broadcasted_iota(jnp.int32, (9 * C, 3 * C), 0)
        s_ids = jax.lax.broadcasted_iota(jnp.int32, (9 * C, 3 * C), 1)
        f_target = (s_ids % C) * 9 + (s_ids // C) * 3
        for dw in range(3):
            q = (f_ids == f_target + dw).astype(jnp.float32)
            l_ref[dw * Cout:(dw + 1) * Cout, :] = jnp.dot(
                wf_ref[:, :], q, preferred_element_type=jnp.float32)
        bcol_ref[:, :] = jnp.transpose(br_ref[:, :], (1, 0))
        xpad_ref[:, 0:PAD] = jnp.zeros((C, PAD), jnp.float32)
        xpad_ref[:, PAD + HW:] = jnp.zeros((C, PAD), jnp.float32)
        z1 = jnp.zeros((1, PAD), jnp.float32)
        cpad_ref[:, 0:PAD] = z1
        cpad_ref[:, PAD + HW:] = z1

    @pl.when(j < NB)
    def _stage_chunk():
        xc = x_ref[0]
        xpad_ref[:, pl.ds(PAD + j * Nb, Nb)] = xc.reshape(C, Nb)
        m = jnp.max(jnp.abs(xc - pi_ref[0]), axis=0)
        cpad_ref[:, pl.ds(PAD + j * Nb, Nb)] = \
            (m > _THRESHOLD).astype(jnp.float32).reshape(1, Nb)

    @pl.when(j > 0)
    def _compute_block():
        rb = j - 1
        lane = jax.lax.broadcasted_iota(jnp.int32, (1, Nb), 1) % W
        mleft = (lane != 0).astype(jnp.float32)
        mright = (lane != (W - 1)).astype(jnp.float32)

        base = PAD + rb * Nb
        for dh in range(3):
            rhs_ref[dh * C:(dh + 1) * C, :] = \
                xpad_ref[:, pl.ds(base + (dh - 1) * W - 128, NW)]

        z = jnp.dot(l_ref[:, :], rhs_ref[:, :],
                    preferred_element_type=jnp.float32)
        y = (jax.lax.slice(z, (0, 127), (Cout, 127 + Nb)) * mleft
             + jax.lax.slice(z, (Cout, 128), (2 * Cout, 128 + Nb))
             + jax.lax.slice(z, (2 * Cout, 129), (3 * Cout, 129 + Nb))
             * mright
             + bcol_ref[:, :])

        dil = None
        for dr in range(3):
            cw = cpad_ref[:, pl.ds(base + (dr - 1) * W - 128, NW)]
            for dc in range(3):
                s = jax.lax.slice(cw, (0, 128 + dc - 1),
                                  (1, 128 + dc - 1 + Nb))
                if dc == 0:
                    s = s * mleft
                elif dc == 2:
                    s = s * mright
                dil = s if dil is None else jnp.maximum(dil, s)

        sel = jnp.where(dil > 0.0, y, po_ref[0].reshape(Cout, Nb))
        for t in range(Rb // 8):
            out_ref[0, :, t * 8:(t + 1) * 8, :] = jax.lax.slice(
                sel, (0, t * 8 * W), (Cout, (t + 1) * 8 * W)
            ).reshape(Cout, 8, W)


def kernel(x, prev_input, prev_output, weight, bias):
    B, C, H, W = x.shape
    Cout = weight.shape[0]
    HW = H * W
    Rb = 64
    NB = H // Rb
    Nb = Rb * W
    PAD = 256

    wf = weight.reshape(Cout, C * _KH * _KW)
    br = bias.reshape(1, Cout)

    body = partial(_cbconv_body, C=C, Cout=Cout, H=H, W=W, Rb=Rb, PAD=PAD,
                   NB=NB)
    last = NB - 1
    out = pl.pallas_call(
        body,
        grid=(B, NB + 1),
        in_specs=[
            pl.BlockSpec((1, C, Rb, W),
                         lambda b, j: (b, 0, jnp.minimum(j, last), 0)),
            pl.BlockSpec((1, C, Rb, W),
                         lambda b, j: (b, 0, jnp.minimum(j, last), 0)),
            pl.BlockSpec((1, Cout, Rb, W),
                         lambda b, j: (b, 0, jnp.maximum(j - 1, 0), 0)),
            pl.BlockSpec((Cout, _KH * _KW * C), lambda b, j: (0, 0)),
            pl.BlockSpec((1, Cout), lambda b, j: (0, 0)),
        ],
        out_specs=pl.BlockSpec((1, Cout, Rb, W),
                               lambda b, j: (b, 0, jnp.maximum(j - 1, 0), 0)),
        out_shape=jax.ShapeDtypeStruct((B, Cout, H, W), jnp.float32),
        scratch_shapes=[
            pltpu.VMEM((C, PAD + HW + PAD), jnp.float32),
            pltpu.VMEM((1, PAD + HW + PAD), jnp.float32),
            pltpu.VMEM((3 * C, Nb + 256), jnp.float32),
            pltpu.VMEM((3 * Cout, 3 * C), jnp.float32),
            pltpu.VMEM((Cout, 1), jnp.float32),
        ],
    )(x, prev_input, prev_output, wf, br)
    return out
